# baseline (speedup 1.0000x reference)
import functools

import jax
import jax.numpy as jnp
from jax.experimental import pallas as pl
from jax.experimental.pallas import tpu as pltpu

_NUM_EMBEDDINGS = 1024
_EMBED_DIM = 768
_NUM_HEADS = 4
_DH = _EMBED_DIM // _NUM_HEADS
_COMMITMENT_COST = 0.25

_BLOCK = 256


_AUG = _DH + 3


def _vq_kernel(x_ref, w_ref, q_ref, codes_ref, loss_ref, b_scr, w2_scr,
               waug_scr):
    @pl.when(pl.program_id(0) == 0)
    def _():
        j = jax.lax.broadcasted_iota(jnp.int32, (_NUM_EMBEDDINGS, 1), 0)
        hi = (j & ~31).astype(jnp.float32)
        lo = (j & 31).astype(jnp.float32)
        ones = jnp.ones((_NUM_EMBEDDINGS, 1), jnp.float32)
        for h in range(_NUM_HEADS):
            wh = w_ref[h]
            b_scr[h] = jnp.sum(wh * wh, axis=1)[None, :]
            w2_scr[h] = wh + wh
            waug_scr[h] = jnp.concatenate([wh, ones, hi, lo], axis=1)

    x = x_ref[...]
    acc = jnp.zeros((), dtype=jnp.float32)
    idx_cols = []
    iota_f = jax.lax.broadcasted_iota(
        jnp.int32, (1, _NUM_EMBEDDINGS), 1).astype(jnp.float32)
    for h in range(_NUM_HEADS):
        xh = x[:, h * _DH:(h + 1) * _DH]
        wh = w_ref[h]
        m2 = jax.lax.dot_general(
            xh, w2_scr[h], (((1,), (1,)), ((), ())),
            preferred_element_type=jnp.float32)
        a = jnp.sum(xh * xh, axis=1, keepdims=True)
        d = (a + b_scr[h]) - m2
        dmin = jnp.min(d, axis=1, keepdims=True)
        eq = d == dmin
        onehot0 = jnp.where(eq, 1.0, 0.0).astype(jnp.float32)
        res = jax.lax.dot_general(
            onehot0, waug_scr[h], (((1,), (0,)), ((), ())),
            preferred_element_type=jnp.float32)
        cnt = res[:, _DH:_DH + 1]
        idx_sum = res[:, _DH + 1:_DH + 2] + res[:, _DH + 2:_DH + 3]

        def _tie_path(_):
            idxf = jnp.min(
                jnp.where(eq, iota_f, jnp.float32(_NUM_EMBEDDINGS)),
                axis=1, keepdims=True)
            onehot = (iota_f == idxf).astype(jnp.float32)
            qh = jax.lax.dot_general(
                onehot, wh, (((1,), (0,)), ((), ())),
                preferred_element_type=jnp.float32)
            return idxf, qh

        def _fast_path(_):
            return idx_sum, res[:, :_DH]

        idxf, qh = jax.lax.cond(
            jnp.max(cnt) > 1.5, _tie_path, _fast_path, 0)
        idx_cols.append(idxf)
        q_ref[:, h * _DH:(h + 1) * _DH] = qh
        acc = acc + jnp.sum(dmin)
    codes_ref[...] = jnp.concatenate(idx_cols, axis=1).astype(jnp.int32)
    loss_ref[...] = acc.reshape(1, 1, 1)


@jax.jit
def kernel(inputs, emb_weights):
    input_shape = inputs.shape
    n = input_shape[0] * input_shape[1]
    x = inputs.reshape(n, _EMBED_DIM)
    nblocks = n // _BLOCK

    q, codes, loss_parts = pl.pallas_call(
        _vq_kernel,
        grid=(nblocks,),
        in_specs=[
            pl.BlockSpec((_BLOCK, _EMBED_DIM), lambda i: (i, 0)),
            pl.BlockSpec((_NUM_HEADS, _NUM_EMBEDDINGS, _DH),
                         lambda i: (0, 0, 0)),
        ],
        out_specs=[
            pl.BlockSpec((_BLOCK, _EMBED_DIM), lambda i: (i, 0)),
            pl.BlockSpec((_BLOCK, _NUM_HEADS), lambda i: (i, 0)),
            pl.BlockSpec((1, 1, 1), lambda i: (i, 0, 0)),
        ],
        out_shape=[
            jax.ShapeDtypeStruct((n, _EMBED_DIM), jnp.float32),
            jax.ShapeDtypeStruct((n, _NUM_HEADS), jnp.int32),
            jax.ShapeDtypeStruct((nblocks, 1, 1), jnp.float32),
        ],
        scratch_shapes=[
            pltpu.VMEM((_NUM_HEADS, 1, _NUM_EMBEDDINGS), jnp.float32),
            pltpu.VMEM((_NUM_HEADS, _NUM_EMBEDDINGS, _DH), jnp.float32),
            pltpu.VMEM((_NUM_HEADS, _NUM_EMBEDDINGS, _AUG), jnp.float32),
        ],
        compiler_params=pltpu.CompilerParams(
            dimension_semantics=("arbitrary",)),
    )(x, emb_weights)

    numel = n * _EMBED_DIM
    loss = jnp.sum(loss_parts) * (_COMMITMENT_COST / numel)
    quantized = q.reshape(input_shape)
    vq_codes = codes.T[:, :, None]
    return loss, quantized, vq_codes
